# baseline (speedup 1.0000x reference)
import jax
import jax.numpy as jnp
from jax import lax
from jax.experimental import pallas as pl
from jax.experimental.pallas import tpu as pltpu
from jax.experimental.pallas import tpu_sc as plsc

VOCAB = 100000
EMBED_DIM = 64
MAX_LENGTH = 200
BATCH = 4096
SEQ = 200

_NC = 2
_NS = 16
_NW = _NC * _NS
_BW = BATCH // _NW
_NBUF = 4
_NTB = 2


def _sc_body(xt_hbm, table_hbm, pos_hbm, out_hbm,
             idx_v, pos_v, rows_v, tps_v, gsems, wsems):
    wid = lax.axis_index("s") * _NC + lax.axis_index("c")
    b0 = wid * _BW

    pltpu.sync_copy(xt_hbm.at[:, pl.ds(b0, _BW)], idx_v)
    pltpu.sync_copy(pos_hbm, pos_v)

    def fetch(j, s):
        pltpu.async_copy(table_hbm.at[idx_v.at[s]], rows_v.at[j], gsems[j])

    def wait_gather(j, s):
        pltpu.make_async_copy(table_hbm.at[idx_v.at[s]],
                              rows_v.at[j], gsems[j]).wait()

    def wait_wb(j):
        pltpu.make_async_copy(tps_v.at[j],
                              out_hbm.at[0, :, pl.ds(b0, _BW)],
                              wsems[j]).wait()

    base_b = [lax.iota(jnp.int32, 16) + bg * 16 for bg in range(_BW // 16)]

    def transpose_add(jg, jt, s):
        idx_s = jnp.full((16,), s, dtype=jnp.int32)

        def body(d, c):
            idx_d = jnp.full((16,), d, dtype=jnp.int32)
            p = plsc.load_gather(pos_v, [idx_s, idx_d])
            for bg in range(_BW // 16):
                v = plsc.load_gather(rows_v.at[jg], [base_b[bg], idx_d])
                tps_v[jt, d, pl.ds(bg * 16, 16)] = v + p
            return c
        lax.fori_loop(0, EMBED_DIM, body, 0)

    for j in range(_NBUF - 1):
        fetch(j, j)

    def outer(k, carry):
        for b in range(_NBUF):
            s = k * _NBUF + b
            jg = b
            jgf = (b + _NBUF - 1) % _NBUF
            jt = b % _NTB

            @pl.when(s + _NBUF - 1 < SEQ)
            def _():
                fetch(jgf, s + _NBUF - 1)

            wait_gather(jg, s)

            @pl.when(s >= _NTB)
            def _():
                wait_wb(jt)

            transpose_add(jg, jt, s)
            pltpu.async_copy(tps_v.at[jt],
                             out_hbm.at[s, :, pl.ds(b0, _BW)], wsems[jt])
        return carry

    lax.fori_loop(0, SEQ // _NBUF, outer, 0)
    wait_wb(0)
    wait_wb(1)


@jax.jit
def kernel(x, table, pos_table):
    def body(x_h, t_h, p_h, o_h, iv, pv, rv, tv,
             g0, g1, g2, g3, w0, w1):
        _sc_body(x_h, t_h, p_h, o_h, iv, pv, rv, tv,
                 (g0, g1, g2, g3), (w0, w1))

    run = pl.kernel(
        body,
        out_type=jax.ShapeDtypeStruct((SEQ, EMBED_DIM, BATCH), jnp.float32),
        mesh=plsc.VectorSubcoreMesh(core_axis_name="c", subcore_axis_name="s"),
        compiler_params=pltpu.CompilerParams(use_tc_tiling_on_sc=False,
                                             needs_layout_passes=False),
        scratch_types=[
            pltpu.VMEM((SEQ, _BW), jnp.int32),
            pltpu.VMEM((MAX_LENGTH, EMBED_DIM), jnp.float32),
            pltpu.VMEM((_NBUF, _BW, EMBED_DIM), jnp.float32),
            pltpu.VMEM((_NTB, EMBED_DIM, _BW), jnp.float32),
        ] + [pltpu.SemaphoreType.DMA] * (_NBUF + _NTB),
    )
    xt = jnp.transpose(x.astype(jnp.int32))
    out_t = run(xt, table, pos_table)
    return jnp.transpose(out_t, (2, 0, 1))


# speedup vs baseline: 4.2164x; 4.2164x over previous
import jax
import jax.numpy as jnp
from jax import lax
from jax.experimental import pallas as pl
from jax.experimental.pallas import tpu as pltpu
from jax.experimental.pallas import tpu_sc as plsc

VOCAB = 100000
EMBED_DIM = 64
MAX_LENGTH = 200
BATCH = 4096
SEQ = 200

_NC = 2
_NS = 16
_NW = _NC * _NS
_BW = BATCH // _NW
_NBUF = 4
_NPART = 2
_SEQ_P = SEQ // _NPART


def _sc_body(xt_hbm, table_hbm, pos_hbm, out_hbm,
             idx_v, pos_v, rows_v, gsems, wsems):
    wid = lax.axis_index("s") * _NC + lax.axis_index("c")
    b0 = wid * _BW

    pltpu.sync_copy(xt_hbm.at[:, pl.ds(b0, _BW)], idx_v)
    pltpu.sync_copy(pos_hbm, pos_v)

    def fetch(j, s):
        pltpu.async_copy(table_hbm.at[idx_v.at[s]], rows_v.at[j], gsems[j])

    def wait_gather(j, s):
        pltpu.make_async_copy(table_hbm.at[idx_v.at[s]],
                              rows_v.at[j], gsems[j]).wait()

    def wait_wb(j):
        pltpu.make_async_copy(rows_v.at[j],
                              out_hbm.at[pl.ds(b0, _BW), 0], wsems[j]).wait()

    def add_pos(j, s):
        pos_q = [pos_v[s, pl.ds(q * 16, 16)] for q in range(EMBED_DIM // 16)]

        def body(i, c):
            for u in range(8):
                t = i * 8 + u
                for q in range(EMBED_DIM // 16):
                    plsc.addupdate(rows_v.at[j, t, pl.ds(q * 16, 16)],
                                   pos_q[q])
            return c
        lax.fori_loop(0, _BW // 8, body, 0)

    for j in range(_NBUF - 1):
        fetch(j, j)

    def outer(k, carry):
        for b in range(_NBUF):
            s = k * _NBUF + b
            jf = (b + _NBUF - 1) % _NBUF

            @pl.when(s >= 1)
            def _():
                wait_wb(jf)

            @pl.when(s + _NBUF - 1 < _SEQ_P)
            def _():
                fetch(jf, s + _NBUF - 1)

            wait_gather(b, s)
            add_pos(b, s)
            pltpu.async_copy(rows_v.at[b],
                             out_hbm.at[pl.ds(b0, _BW), s], wsems[b])
        return carry

    lax.fori_loop(0, _SEQ_P // _NBUF, outer, 0)
    wait_wb((_SEQ_P - 1) % _NBUF)


_PAIRS = _SEQ_P // 2


def _tc_body(in_ref, out_ref):
    v = in_ref[...].reshape(_BW, _PAIRS, 128)
    for k in range(_PAIRS):
        t = v[:, k, :].T
        out_ref[2 * k, :, :] = t[0:EMBED_DIM, :]
        out_ref[2 * k + 1, :, :] = t[EMBED_DIM:, :]


def _tc_transpose(part, prev, s_blk):
    in2 = part.reshape(BATCH * _PAIRS, 128)
    kwargs = {}
    operands = [in2]
    if prev is not None:
        operands.append(prev)
        kwargs["input_output_aliases"] = {1: 0}
    return pl.pallas_call(
        (lambda i_ref, p_ref, o_ref: _tc_body(i_ref, o_ref)) if prev is not None
        else _tc_body,
        grid=(BATCH // _BW,),
        in_specs=[pl.BlockSpec((_BW * _PAIRS, 128), lambda b: (b, 0))]
        + ([pl.BlockSpec(memory_space=pltpu.MemorySpace.HBM)]
           if prev is not None else []),
        out_specs=pl.BlockSpec((_SEQ_P, EMBED_DIM, _BW),
                               lambda b, _s=s_blk: (_s, 0, b)),
        out_shape=jax.ShapeDtypeStruct((SEQ, EMBED_DIM, BATCH), jnp.float32),
        **kwargs,
    )(*operands)


@jax.jit
def kernel(x, table, pos_table):
    def body(x_h, t_h, p_h, o_h, iv, pv, rv,
             g0, g1, g2, g3, w0, w1, w2, w3):
        _sc_body(x_h, t_h, p_h, o_h, iv, pv, rv,
                 (g0, g1, g2, g3), (w0, w1, w2, w3))

    run = pl.kernel(
        body,
        out_type=jax.ShapeDtypeStruct((BATCH, _SEQ_P, EMBED_DIM), jnp.float32),
        mesh=plsc.VectorSubcoreMesh(core_axis_name="c", subcore_axis_name="s"),
        compiler_params=pltpu.CompilerParams(use_tc_tiling_on_sc=False),
        scratch_types=[
            pltpu.VMEM((_SEQ_P, _BW), jnp.int32),
            pltpu.VMEM((_SEQ_P, EMBED_DIM), jnp.float32),
            pltpu.VMEM((_NBUF, _BW, EMBED_DIM), jnp.float32),
        ] + [pltpu.SemaphoreType.DMA] * (2 * _NBUF),
    )
    xt = jnp.transpose(x.astype(jnp.int32))
    out_t = None
    for p in range(_NPART):
        xt_p = lax.slice_in_dim(xt, p * _SEQ_P, (p + 1) * _SEQ_P, axis=0)
        pos_p = lax.slice_in_dim(pos_table, p * _SEQ_P, (p + 1) * _SEQ_P,
                                 axis=0)
        part = run(xt_p, table, pos_p)
        out_t = _tc_transpose(part, out_t, p)
    return jnp.transpose(out_t, (2, 0, 1))
